# baseline (speedup 1.0000x reference)
import functools

import jax
import jax.numpy as jnp
from jax import lax
from jax.experimental import pallas as pl
from jax.experimental.pallas import tpu as pltpu
from jax.experimental.pallas import tpu_sc as plsc

B = 16384
CAT = 26
CONT = 13
D = 64
V = 100000
PITCH = 128

NC, NS = 2, 16
NW = NC * NS
RPW = B // NW
IPW = RPW * CAT
IDXW = 128
NROW = IPW // IDXW
GQ = 128

CTAB = 32
VPAD = 100096
COLS_BLK = 2944


def _tc_body(wcat_ref, embt_ref, scores_ref):
    scores_ref[...] = jnp.dot(wcat_ref[...], embt_ref[...],
                              preferred_element_type=jnp.float32)


_tc_call = pl.pallas_call(
    _tc_body,
    grid=(VPAD // COLS_BLK,),
    in_specs=[
        pl.BlockSpec((CTAB, D), lambda i: (0, 0)),
        pl.BlockSpec((D, COLS_BLK), lambda i: (0, i)),
    ],
    out_specs=pl.BlockSpec((CTAB, COLS_BLK), lambda i: (0, i)),
    out_shape=jax.ShapeDtypeStruct((CTAB, VPAD), jnp.float32),
)


def _sc_body(scores_hbm, fidx_hbm, cont_hbm, wb_hbm, out_hbm,
             idx_v, g_v, cont_v, wb_v, out_v, sem):
    w = lax.axis_index("s") * NC + lax.axis_index("c")
    base = w * RPW

    pltpu.sync_copy(fidx_hbm.at[:, pl.ds(base, RPW)], idx_v)
    for c in range(CAT):
        for q in range(RPW // GQ):
            pltpu.async_copy(
                scores_hbm.at[idx_v.at[c, pl.ds(q * GQ, GQ)]],
                g_v.at[c, q], sem)

    pltpu.sync_copy(cont_hbm.at[:, pl.ds(base, RPW)], cont_v)
    pltpu.sync_copy(wb_hbm, wb_v)
    wrow = [wb_v[j, :] for j in range(CONT + 1)]
    for m in range(RPW // 16):
        r0 = m * 16
        acc = wrow[CONT]
        for j in range(CONT):
            acc = acc + cont_v[j, pl.ds(r0, 16)] * wrow[j]
        out_v[pl.ds(r0, 16)] = acc

    for c in range(CAT):
        for q in range(RPW // GQ):
            pltpu.make_async_copy(
                scores_hbm.at[idx_v.at[c, pl.ds(q * GQ, GQ)]],
                g_v.at[c, q], sem).wait()
    for m in range(RPW // 16):
        r0 = m * 16
        q, p = divmod(r0, GQ)
        acc = out_v[pl.ds(r0, 16)]
        for c in range(CAT):
            acc = acc + g_v[c, q, pl.ds(p, 16)]
        out_v[pl.ds(r0, 16)] = 1.0 / (1.0 + jnp.exp(-acc))

    pltpu.sync_copy(out_v, out_hbm.at[pl.ds(base, RPW)])


_sc_call = functools.partial(
    pl.kernel,
    out_type=jax.ShapeDtypeStruct((B,), jnp.float32),
    mesh=plsc.VectorSubcoreMesh(core_axis_name="c", subcore_axis_name="s"),
    compiler_params=pltpu.CompilerParams(needs_layout_passes=False),
    scratch_types=[
        pltpu.VMEM((CAT, RPW), jnp.int32),
        pltpu.VMEM((CAT, RPW // GQ, GQ), jnp.float32),
        pltpu.VMEM((CONT, RPW), jnp.float32),
        pltpu.VMEM((CONT + 1, 16), jnp.float32),
        pltpu.VMEM((RPW,), jnp.float32),
        pltpu.SemaphoreType.DMA,
    ],
)(_sc_body)


def kernel(categorical_features, continous_features, emb_table, W, b):
    wcat = W[:CAT * D].reshape(CAT, D)
    wcat32 = jnp.zeros((CTAB, D), jnp.float32).at[:CAT].set(wcat)
    scores = _tc_call(wcat32, emb_table.T)

    fidx = categorical_features.T.astype(jnp.int32) + jnp.arange(
        CAT, dtype=jnp.int32)[:, None] * VPAD

    wb = jnp.concatenate([W[CAT * D:, 0], b]).astype(jnp.float32)
    wb = jnp.broadcast_to(wb[:, None], (CONT + 1, 16))

    out = _sc_call(scores.reshape(-1), fidx, continous_features.T, wb)
    return out.reshape(B, 1)


# speedup vs baseline: 1.1105x; 1.1105x over previous
import functools

import jax
import jax.numpy as jnp
from jax import lax
from jax.experimental import pallas as pl
from jax.experimental.pallas import tpu as pltpu
from jax.experimental.pallas import tpu_sc as plsc

B = 16384
CAT = 26
CONT = 13
D = 64
V = 100000
PITCH = 128

NC, NS = 2, 16
NW = NC * NS
RPW = B // NW
IPW = RPW * CAT
IDXW = 128
NROW = IPW // IDXW
GQ = 128

CTAB = 32
VPAD = 100096
COLS_BLK = 5888


def _tc_body(wcat_ref, embt_ref, scores_ref):
    scores_ref[...] = jnp.dot(wcat_ref[...], embt_ref[...],
                              preferred_element_type=jnp.float32)


_tc_call = pl.pallas_call(
    _tc_body,
    grid=(VPAD // COLS_BLK,),
    in_specs=[
        pl.BlockSpec((CTAB, D), lambda i: (0, 0)),
        pl.BlockSpec((D, COLS_BLK), lambda i: (0, i)),
    ],
    out_specs=pl.BlockSpec((CTAB, COLS_BLK), lambda i: (0, i)),
    out_shape=jax.ShapeDtypeStruct((CTAB, VPAD), jnp.float32),
)


def _sc_body(scores_hbm, fidx_hbm, cont_hbm, wb_hbm, out_hbm,
             idx_v, g_v, cont_v, wb_v, out_v, sem):
    w = lax.axis_index("s") * NC + lax.axis_index("c")
    base = w * RPW

    pltpu.sync_copy(fidx_hbm.at[:, pl.ds(base, RPW)], idx_v)
    for c in range(CAT):
        for q in range(RPW // GQ):
            pltpu.async_copy(
                scores_hbm.at[idx_v.at[c, pl.ds(q * GQ, GQ)]],
                g_v.at[c, q], sem)

    pltpu.sync_copy(cont_hbm.at[:, pl.ds(base, RPW)], cont_v)
    pltpu.sync_copy(wb_hbm, wb_v)
    wrow = [wb_v[j, :] for j in range(CONT + 1)]
    for m in range(RPW // 16):
        r0 = m * 16
        acc = wrow[CONT]
        for j in range(CONT):
            acc = acc + cont_v[j, pl.ds(r0, 16)] * wrow[j]
        out_v[pl.ds(r0, 16)] = acc

    for c in range(CAT):
        for q in range(RPW // GQ):
            pltpu.make_async_copy(
                scores_hbm.at[idx_v.at[c, pl.ds(q * GQ, GQ)]],
                g_v.at[c, q], sem).wait()
    for m in range(RPW // 16):
        r0 = m * 16
        q, p = divmod(r0, GQ)
        acc = out_v[pl.ds(r0, 16)]
        for c in range(CAT):
            acc = acc + g_v[c, q, pl.ds(p, 16)]
        out_v[pl.ds(r0, 16)] = 1.0 / (1.0 + jnp.exp(-acc))

    pltpu.sync_copy(out_v, out_hbm.at[pl.ds(base, RPW)])


_sc_call = functools.partial(
    pl.kernel,
    out_type=jax.ShapeDtypeStruct((B,), jnp.float32),
    mesh=plsc.VectorSubcoreMesh(core_axis_name="c", subcore_axis_name="s"),
    compiler_params=pltpu.CompilerParams(needs_layout_passes=False),
    scratch_types=[
        pltpu.VMEM((CAT, RPW), jnp.int32),
        pltpu.VMEM((CAT, RPW // GQ, GQ), jnp.float32),
        pltpu.VMEM((CONT, RPW), jnp.float32),
        pltpu.VMEM((CONT + 1, 16), jnp.float32),
        pltpu.VMEM((RPW,), jnp.float32),
        pltpu.SemaphoreType.DMA,
    ],
)(_sc_body)


def kernel(categorical_features, continous_features, emb_table, W, b):
    wcat = W[:CAT * D].reshape(CAT, D)
    wcat32 = jnp.zeros((CTAB, D), jnp.float32).at[:CAT].set(wcat)
    scores = _tc_call(wcat32, emb_table.T)

    fidx = categorical_features.T.astype(jnp.int32) + jnp.arange(
        CAT, dtype=jnp.int32)[:, None] * VPAD

    wb = jnp.concatenate([W[CAT * D:, 0], b]).astype(jnp.float32)
    wb = jnp.broadcast_to(wb[:, None], (CONT + 1, 16))

    out = _sc_call(scores.reshape(-1), fidx, continous_features.T, wb)
    return out.reshape(B, 1)


# speedup vs baseline: 1.1682x; 1.0519x over previous
import functools

import jax
import jax.numpy as jnp
from jax import lax
from jax.experimental import pallas as pl
from jax.experimental.pallas import tpu as pltpu
from jax.experimental.pallas import tpu_sc as plsc

B = 16384
CAT = 26
CONT = 13
D = 64
V = 100000
PITCH = 128

NC, NS = 2, 16
NW = NC * NS
RPW = B // NW
IPW = RPW * CAT
IDXW = 128
NROW = IPW // IDXW
GQ = 128

CTAB = 32
VPAD = 100096
COLS_BLK = 8704


def _tc_body(wcat_ref, embt_ref, scores_ref):
    scores_ref[...] = jnp.dot(wcat_ref[...], embt_ref[...],
                              preferred_element_type=jnp.float32)


_tc_call = pl.pallas_call(
    _tc_body,
    grid=((VPAD + COLS_BLK - 1) // COLS_BLK,),
    in_specs=[
        pl.BlockSpec((CTAB, D), lambda i: (0, 0)),
        pl.BlockSpec((D, COLS_BLK), lambda i: (0, i)),
    ],
    out_specs=pl.BlockSpec((CTAB, COLS_BLK), lambda i: (0, i)),
    out_shape=jax.ShapeDtypeStruct((CTAB, VPAD), jnp.float32),
)


def _sc_body(scores_hbm, fidx_hbm, cont_hbm, wb_hbm, out_hbm,
             idx_v, g_v, cont_v, wb_v, out_v, sem):
    w = lax.axis_index("s") * NC + lax.axis_index("c")
    base = w * RPW

    pltpu.sync_copy(fidx_hbm.at[:, pl.ds(base, RPW)], idx_v)
    for c in range(CAT):
        for q in range(RPW // GQ):
            pltpu.async_copy(
                scores_hbm.at[idx_v.at[c, pl.ds(q * GQ, GQ)]],
                g_v.at[c, q], sem)

    pltpu.sync_copy(cont_hbm.at[:, pl.ds(base, RPW)], cont_v)
    pltpu.sync_copy(wb_hbm, wb_v)
    wrow = [wb_v[j, :] for j in range(CONT + 1)]
    for m in range(RPW // 16):
        r0 = m * 16
        acc = wrow[CONT]
        for j in range(CONT):
            acc = acc + cont_v[j, pl.ds(r0, 16)] * wrow[j]
        out_v[pl.ds(r0, 16)] = acc

    for c in range(CAT):
        for q in range(RPW // GQ):
            pltpu.make_async_copy(
                scores_hbm.at[idx_v.at[c, pl.ds(q * GQ, GQ)]],
                g_v.at[c, q], sem).wait()
    for m in range(RPW // 16):
        r0 = m * 16
        q, p = divmod(r0, GQ)
        acc = out_v[pl.ds(r0, 16)]
        for c in range(CAT):
            acc = acc + g_v[c, q, pl.ds(p, 16)]
        out_v[pl.ds(r0, 16)] = 1.0 / (1.0 + jnp.exp(-acc))

    pltpu.sync_copy(out_v, out_hbm.at[pl.ds(base, RPW)])


_sc_call = functools.partial(
    pl.kernel,
    out_type=jax.ShapeDtypeStruct((B,), jnp.float32),
    mesh=plsc.VectorSubcoreMesh(core_axis_name="c", subcore_axis_name="s"),
    compiler_params=pltpu.CompilerParams(needs_layout_passes=False),
    scratch_types=[
        pltpu.VMEM((CAT, RPW), jnp.int32),
        pltpu.VMEM((CAT, RPW // GQ, GQ), jnp.float32),
        pltpu.VMEM((CONT, RPW), jnp.float32),
        pltpu.VMEM((CONT + 1, 16), jnp.float32),
        pltpu.VMEM((RPW,), jnp.float32),
        pltpu.SemaphoreType.DMA,
    ],
)(_sc_body)


def kernel(categorical_features, continous_features, emb_table, W, b):
    wcat = W[:CAT * D].reshape(CAT, D)
    wcat32 = jnp.zeros((CTAB, D), jnp.float32).at[:CAT].set(wcat)
    scores = _tc_call(wcat32, emb_table.T)

    fidx = categorical_features.T.astype(jnp.int32) + jnp.arange(
        CAT, dtype=jnp.int32)[:, None] * VPAD

    wb = jnp.concatenate([W[CAT * D:, 0], b]).astype(jnp.float32)
    wb = jnp.broadcast_to(wb[:, None], (CONT + 1, 16))

    out = _sc_call(scores.reshape(-1), fidx, continous_features.T, wb)
    return out.reshape(B, 1)


# speedup vs baseline: 1.1964x; 1.0241x over previous
import functools

import jax
import jax.numpy as jnp
from jax import lax
from jax.experimental import pallas as pl
from jax.experimental.pallas import tpu as pltpu
from jax.experimental.pallas import tpu_sc as plsc

B = 16384
CAT = 26
CONT = 13
D = 64
V = 100000
PITCH = 128

NC, NS = 2, 16
NW = NC * NS
RPW = B // NW
IPW = RPW * CAT
IDXW = 128
NROW = IPW // IDXW
GQ = 128

CTAB = 32
VPAD = 100096
COLS_BLK = 12544


def _tc_body(wcat_ref, embt_ref, scores_ref):
    scores_ref[...] = jnp.dot(wcat_ref[...], embt_ref[...],
                              preferred_element_type=jnp.float32)


_tc_call = pl.pallas_call(
    _tc_body,
    grid=((VPAD + COLS_BLK - 1) // COLS_BLK,),
    in_specs=[
        pl.BlockSpec((CTAB, D), lambda i: (0, 0)),
        pl.BlockSpec((D, COLS_BLK), lambda i: (0, i)),
    ],
    out_specs=pl.BlockSpec((CTAB, COLS_BLK), lambda i: (0, i)),
    out_shape=jax.ShapeDtypeStruct((CTAB, VPAD), jnp.float32),
)


def _sc_body(scores_hbm, fidx_hbm, cont_hbm, wb_hbm, out_hbm,
             idx_v, g_v, cont_v, wb_v, out_v, sem):
    w = lax.axis_index("s") * NC + lax.axis_index("c")
    base = w * RPW

    pltpu.sync_copy(fidx_hbm.at[:, pl.ds(base, RPW)], idx_v)
    for c in range(CAT):
        for q in range(RPW // GQ):
            pltpu.async_copy(
                scores_hbm.at[idx_v.at[c, pl.ds(q * GQ, GQ)]],
                g_v.at[c, q], sem)

    pltpu.sync_copy(cont_hbm.at[:, pl.ds(base, RPW)], cont_v)
    pltpu.sync_copy(wb_hbm, wb_v)
    wrow = [wb_v[j, :] for j in range(CONT + 1)]
    for m in range(RPW // 16):
        r0 = m * 16
        acc = wrow[CONT]
        for j in range(CONT):
            acc = acc + cont_v[j, pl.ds(r0, 16)] * wrow[j]
        out_v[pl.ds(r0, 16)] = acc

    for c in range(CAT):
        for q in range(RPW // GQ):
            pltpu.make_async_copy(
                scores_hbm.at[idx_v.at[c, pl.ds(q * GQ, GQ)]],
                g_v.at[c, q], sem).wait()
    for m in range(RPW // 16):
        r0 = m * 16
        q, p = divmod(r0, GQ)
        acc = out_v[pl.ds(r0, 16)]
        for c in range(CAT):
            acc = acc + g_v[c, q, pl.ds(p, 16)]
        out_v[pl.ds(r0, 16)] = 1.0 / (1.0 + jnp.exp(-acc))

    pltpu.sync_copy(out_v, out_hbm.at[pl.ds(base, RPW)])


_sc_call = functools.partial(
    pl.kernel,
    out_type=jax.ShapeDtypeStruct((B,), jnp.float32),
    mesh=plsc.VectorSubcoreMesh(core_axis_name="c", subcore_axis_name="s"),
    compiler_params=pltpu.CompilerParams(needs_layout_passes=False),
    scratch_types=[
        pltpu.VMEM((CAT, RPW), jnp.int32),
        pltpu.VMEM((CAT, RPW // GQ, GQ), jnp.float32),
        pltpu.VMEM((CONT, RPW), jnp.float32),
        pltpu.VMEM((CONT + 1, 16), jnp.float32),
        pltpu.VMEM((RPW,), jnp.float32),
        pltpu.SemaphoreType.DMA,
    ],
)(_sc_body)


def kernel(categorical_features, continous_features, emb_table, W, b):
    wcat = W[:CAT * D].reshape(CAT, D)
    wcat32 = jnp.zeros((CTAB, D), jnp.float32).at[:CAT].set(wcat)
    scores = _tc_call(wcat32, emb_table.T)

    fidx = categorical_features.T.astype(jnp.int32) + jnp.arange(
        CAT, dtype=jnp.int32)[:, None] * VPAD

    wb = jnp.concatenate([W[CAT * D:, 0], b]).astype(jnp.float32)
    wb = jnp.broadcast_to(wb[:, None], (CONT + 1, 16))

    out = _sc_call(scores.reshape(-1), fidx, continous_features.T, wb)
    return out.reshape(B, 1)


# speedup vs baseline: 1.2074x; 1.0092x over previous
import functools

import jax
import jax.numpy as jnp
from jax import lax
from jax.experimental import pallas as pl
from jax.experimental.pallas import tpu as pltpu
from jax.experimental.pallas import tpu_sc as plsc

B = 16384
CAT = 26
CONT = 13
D = 64
V = 100000
PITCH = 128

NC, NS = 2, 16
NW = NC * NS
RPW = B // NW
IPW = RPW * CAT
IDXW = 128
NROW = IPW // IDXW
GQ = 128

CTAB = 32
VPAD = 100096
COLS_BLK = 20096


def _tc_body(wcat_ref, embt_ref, scores_ref):
    scores_ref[...] = jnp.dot(wcat_ref[...], embt_ref[...],
                              preferred_element_type=jnp.float32)


_tc_call = pl.pallas_call(
    _tc_body,
    grid=((VPAD + COLS_BLK - 1) // COLS_BLK,),
    in_specs=[
        pl.BlockSpec((CTAB, D), lambda i: (0, 0)),
        pl.BlockSpec((D, COLS_BLK), lambda i: (0, i)),
    ],
    out_specs=pl.BlockSpec((CTAB, COLS_BLK), lambda i: (0, i)),
    out_shape=jax.ShapeDtypeStruct((CTAB, VPAD), jnp.float32),
)


def _sc_body(scores_hbm, fidx_hbm, cont_hbm, wb_hbm, out_hbm,
             idx_v, g_v, cont_v, wb_v, out_v, sem):
    w = lax.axis_index("s") * NC + lax.axis_index("c")
    base = w * RPW

    pltpu.sync_copy(fidx_hbm.at[:, pl.ds(base, RPW)], idx_v)
    for c in range(CAT):
        for q in range(RPW // GQ):
            pltpu.async_copy(
                scores_hbm.at[idx_v.at[c, pl.ds(q * GQ, GQ)]],
                g_v.at[c, q], sem)

    pltpu.sync_copy(cont_hbm.at[:, pl.ds(base, RPW)], cont_v)
    pltpu.sync_copy(wb_hbm, wb_v)
    wrow = [wb_v[j, :] for j in range(CONT + 1)]
    for m in range(RPW // 16):
        r0 = m * 16
        acc = wrow[CONT]
        for j in range(CONT):
            acc = acc + cont_v[j, pl.ds(r0, 16)] * wrow[j]
        out_v[pl.ds(r0, 16)] = acc

    for c in range(CAT):
        for q in range(RPW // GQ):
            pltpu.make_async_copy(
                scores_hbm.at[idx_v.at[c, pl.ds(q * GQ, GQ)]],
                g_v.at[c, q], sem).wait()
    for m in range(RPW // 16):
        r0 = m * 16
        q, p = divmod(r0, GQ)
        acc = out_v[pl.ds(r0, 16)]
        for c in range(CAT):
            acc = acc + g_v[c, q, pl.ds(p, 16)]
        out_v[pl.ds(r0, 16)] = 1.0 / (1.0 + jnp.exp(-acc))

    pltpu.sync_copy(out_v, out_hbm.at[pl.ds(base, RPW)])


_sc_call = functools.partial(
    pl.kernel,
    out_type=jax.ShapeDtypeStruct((B,), jnp.float32),
    mesh=plsc.VectorSubcoreMesh(core_axis_name="c", subcore_axis_name="s"),
    compiler_params=pltpu.CompilerParams(needs_layout_passes=False),
    scratch_types=[
        pltpu.VMEM((CAT, RPW), jnp.int32),
        pltpu.VMEM((CAT, RPW // GQ, GQ), jnp.float32),
        pltpu.VMEM((CONT, RPW), jnp.float32),
        pltpu.VMEM((CONT + 1, 16), jnp.float32),
        pltpu.VMEM((RPW,), jnp.float32),
        pltpu.SemaphoreType.DMA,
    ],
)(_sc_body)


def kernel(categorical_features, continous_features, emb_table, W, b):
    wcat = W[:CAT * D].reshape(CAT, D)
    wcat32 = jnp.zeros((CTAB, D), jnp.float32).at[:CAT].set(wcat)
    scores = _tc_call(wcat32, emb_table.T)

    fidx = categorical_features.T.astype(jnp.int32) + jnp.arange(
        CAT, dtype=jnp.int32)[:, None] * VPAD

    wb = jnp.concatenate([W[CAT * D:, 0], b]).astype(jnp.float32)
    wb = jnp.broadcast_to(wb[:, None], (CONT + 1, 16))

    out = _sc_call(scores.reshape(-1), fidx, continous_features.T, wb)
    return out.reshape(B, 1)


# speedup vs baseline: 1.2179x; 1.0087x over previous
import functools

import jax
import jax.numpy as jnp
from jax import lax
from jax.experimental import pallas as pl
from jax.experimental.pallas import tpu as pltpu
from jax.experimental.pallas import tpu_sc as plsc

B = 16384
CAT = 26
CONT = 13
D = 64
V = 100000
PITCH = 128

NC, NS = 2, 16
NW = NC * NS
RPW = B // NW
IPW = RPW * CAT
IDXW = 128
NROW = IPW // IDXW
GQ = 128

CTAB = CAT
VPAD = 100096
COLS_BLK = 20096


def _tc_body(wcat_ref, embt_ref, scores_ref):
    scores_ref[...] = jnp.dot(wcat_ref[...], embt_ref[...],
                              preferred_element_type=jnp.float32)


_tc_call = pl.pallas_call(
    _tc_body,
    grid=((VPAD + COLS_BLK - 1) // COLS_BLK,),
    in_specs=[
        pl.BlockSpec((CTAB, D), lambda i: (0, 0)),
        pl.BlockSpec((D, COLS_BLK), lambda i: (0, i)),
    ],
    out_specs=pl.BlockSpec((CTAB, COLS_BLK), lambda i: (0, i)),
    out_shape=jax.ShapeDtypeStruct((CTAB, VPAD), jnp.float32),
)


def _sc_body(scores_hbm, fidx_hbm, cont_hbm, wb_hbm, out_hbm,
             idx_v, g_v, cont_v, wb_v, out_v, sem):
    w = lax.axis_index("s") * NC + lax.axis_index("c")
    base = w * RPW

    pltpu.sync_copy(fidx_hbm.at[:, pl.ds(base, RPW)], idx_v)
    for c in range(CAT):
        for q in range(RPW // GQ):
            pltpu.async_copy(
                scores_hbm.at[idx_v.at[c, pl.ds(q * GQ, GQ)]],
                g_v.at[c, q], sem)

    pltpu.sync_copy(cont_hbm.at[:, pl.ds(base, RPW)], cont_v)
    pltpu.sync_copy(wb_hbm, wb_v)
    wrow = [wb_v[j, :] for j in range(CONT + 1)]
    for m in range(RPW // 16):
        r0 = m * 16
        acc = wrow[CONT]
        for j in range(CONT):
            acc = acc + cont_v[j, pl.ds(r0, 16)] * wrow[j]
        out_v[pl.ds(r0, 16)] = acc

    for c in range(CAT):
        for q in range(RPW // GQ):
            pltpu.make_async_copy(
                scores_hbm.at[idx_v.at[c, pl.ds(q * GQ, GQ)]],
                g_v.at[c, q], sem).wait()
    for m in range(RPW // 16):
        r0 = m * 16
        q, p = divmod(r0, GQ)
        acc = out_v[pl.ds(r0, 16)]
        for c in range(CAT):
            acc = acc + g_v[c, q, pl.ds(p, 16)]
        out_v[pl.ds(r0, 16)] = 1.0 / (1.0 + jnp.exp(-acc))

    pltpu.sync_copy(out_v, out_hbm.at[pl.ds(base, RPW)])


_sc_call = functools.partial(
    pl.kernel,
    out_type=jax.ShapeDtypeStruct((B,), jnp.float32),
    mesh=plsc.VectorSubcoreMesh(core_axis_name="c", subcore_axis_name="s"),
    compiler_params=pltpu.CompilerParams(needs_layout_passes=False),
    scratch_types=[
        pltpu.VMEM((CAT, RPW), jnp.int32),
        pltpu.VMEM((CAT, RPW // GQ, GQ), jnp.float32),
        pltpu.VMEM((CONT, RPW), jnp.float32),
        pltpu.VMEM((CONT + 1, 16), jnp.float32),
        pltpu.VMEM((RPW,), jnp.float32),
        pltpu.SemaphoreType.DMA,
    ],
)(_sc_body)


def kernel(categorical_features, continous_features, emb_table, W, b):
    wcat = W[:CAT * D].reshape(CAT, D)
    scores = _tc_call(wcat, emb_table.T)

    fidx = categorical_features.T.astype(jnp.int32) + jnp.arange(
        CAT, dtype=jnp.int32)[:, None] * VPAD

    wb = jnp.concatenate([W[CAT * D:, 0], b]).astype(jnp.float32)
    wb = jnp.broadcast_to(wb[:, None], (CONT + 1, 16))

    out = _sc_call(scores.reshape(-1), fidx, continous_features.T, wb)
    return out.reshape(B, 1)


# speedup vs baseline: 1.2238x; 1.0048x over previous
import functools

import jax
import jax.numpy as jnp
from jax import lax
from jax.experimental import pallas as pl
from jax.experimental.pallas import tpu as pltpu
from jax.experimental.pallas import tpu_sc as plsc

B = 16384
CAT = 26
CONT = 13
D = 64
V = 100000
PITCH = 128

NC, NS = 2, 16
NW = NC * NS
RPW = B // NW
IPW = RPW * CAT
IDXW = 128
NROW = IPW // IDXW
GQ = 128

CTAB = CAT
VPAD = 100096
COLS_BLK = 25088


def _tc_body(wcat_ref, embt_ref, scores_ref):
    scores_ref[...] = jnp.dot(wcat_ref[...], embt_ref[...],
                              preferred_element_type=jnp.float32)


_tc_call = pl.pallas_call(
    _tc_body,
    grid=((VPAD + COLS_BLK - 1) // COLS_BLK,),
    in_specs=[
        pl.BlockSpec((CTAB, D), lambda i: (0, 0)),
        pl.BlockSpec((D, COLS_BLK), lambda i: (0, i)),
    ],
    out_specs=pl.BlockSpec((CTAB, COLS_BLK), lambda i: (0, i)),
    out_shape=jax.ShapeDtypeStruct((CTAB, VPAD), jnp.float32),
)


def _sc_body(scores_hbm, fidx_hbm, cont_hbm, wb_hbm, out_hbm,
             idx_v, g_v, cont_v, wb_v, out_v, sem):
    w = lax.axis_index("s") * NC + lax.axis_index("c")
    base = w * RPW

    pltpu.sync_copy(fidx_hbm.at[:, pl.ds(base, RPW)], idx_v)
    for c in range(CAT):
        for q in range(RPW // GQ):
            pltpu.async_copy(
                scores_hbm.at[idx_v.at[c, pl.ds(q * GQ, GQ)]],
                g_v.at[c, q], sem)

    pltpu.sync_copy(cont_hbm.at[:, pl.ds(base, RPW)], cont_v)
    pltpu.sync_copy(wb_hbm, wb_v)
    wrow = [wb_v[j, :] for j in range(CONT + 1)]
    for m in range(RPW // 16):
        r0 = m * 16
        acc = wrow[CONT]
        for j in range(CONT):
            acc = acc + cont_v[j, pl.ds(r0, 16)] * wrow[j]
        out_v[pl.ds(r0, 16)] = acc

    for c in range(CAT):
        for q in range(RPW // GQ):
            pltpu.make_async_copy(
                scores_hbm.at[idx_v.at[c, pl.ds(q * GQ, GQ)]],
                g_v.at[c, q], sem).wait()
    for m in range(RPW // 16):
        r0 = m * 16
        q, p = divmod(r0, GQ)
        acc = out_v[pl.ds(r0, 16)]
        for c in range(CAT):
            acc = acc + g_v[c, q, pl.ds(p, 16)]
        out_v[pl.ds(r0, 16)] = 1.0 / (1.0 + jnp.exp(-acc))

    pltpu.sync_copy(out_v, out_hbm.at[pl.ds(base, RPW)])


_sc_call = functools.partial(
    pl.kernel,
    out_type=jax.ShapeDtypeStruct((B,), jnp.float32),
    mesh=plsc.VectorSubcoreMesh(core_axis_name="c", subcore_axis_name="s"),
    compiler_params=pltpu.CompilerParams(needs_layout_passes=False),
    scratch_types=[
        pltpu.VMEM((CAT, RPW), jnp.int32),
        pltpu.VMEM((CAT, RPW // GQ, GQ), jnp.float32),
        pltpu.VMEM((CONT, RPW), jnp.float32),
        pltpu.VMEM((CONT + 1, 16), jnp.float32),
        pltpu.VMEM((RPW,), jnp.float32),
        pltpu.SemaphoreType.DMA,
    ],
)(_sc_body)


def kernel(categorical_features, continous_features, emb_table, W, b):
    wcat = W[:CAT * D].reshape(CAT, D)
    scores = _tc_call(wcat, emb_table.T)

    fidx = categorical_features.T.astype(jnp.int32) + jnp.arange(
        CAT, dtype=jnp.int32)[:, None] * VPAD

    wb = jnp.concatenate([W[CAT * D:, 0], b]).astype(jnp.float32)
    wb = jnp.broadcast_to(wb[:, None], (CONT + 1, 16))

    out = _sc_call(scores.reshape(-1), fidx, continous_features.T, wb)
    return out.reshape(B, 1)


# speedup vs baseline: 1.2263x; 1.0021x over previous
import functools

import jax
import jax.numpy as jnp
from jax import lax
from jax.experimental import pallas as pl
from jax.experimental.pallas import tpu as pltpu
from jax.experimental.pallas import tpu_sc as plsc

B = 16384
CAT = 26
CONT = 13
D = 64
V = 100000

NC, NS = 2, 16
NW = NC * NS
RPW = B // NW
GQ = 128

CTAB = CAT
VPAD = 100096
COLS_BLK = 25088


def _tc_body(wcat_ref, embt_ref, scores_ref):
    scores_ref[...] = jnp.dot(wcat_ref[...], embt_ref[...],
                              preferred_element_type=jnp.float32)


_tc_call = pl.pallas_call(
    _tc_body,
    grid=((VPAD + COLS_BLK - 1) // COLS_BLK,),
    in_specs=[
        pl.BlockSpec((CTAB, D), lambda i: (0, 0)),
        pl.BlockSpec((D, COLS_BLK), lambda i: (0, i)),
    ],
    out_specs=pl.BlockSpec((CTAB, COLS_BLK), lambda i: (0, i)),
    out_shape=jax.ShapeDtypeStruct((CTAB, VPAD), jnp.float32),
)


def _sc_body(scores_hbm, fidx_hbm, cont_hbm, wb_hbm, out_hbm,
             idx_v, g_v, cont_v, wb_v, out_v, sem):
    w = lax.axis_index("s") * NC + lax.axis_index("c")
    base = w * RPW

    pltpu.sync_copy(fidx_hbm.at[:, pl.ds(base, RPW)], idx_v)
    for c in range(CAT):
        for q in range(RPW // GQ):
            pltpu.async_copy(
                scores_hbm.at[idx_v.at[c, pl.ds(q * GQ, GQ)]],
                g_v.at[c, q], sem)

    pltpu.sync_copy(cont_hbm.at[:, pl.ds(base, RPW)], cont_v)
    pltpu.sync_copy(wb_hbm, wb_v)
    wrow = [wb_v[j, :] for j in range(CONT + 1)]
    for m in range(RPW // 16):
        r0 = m * 16
        acc = wrow[CONT]
        for j in range(CONT):
            acc = acc + cont_v[j, pl.ds(r0, 16)] * wrow[j]
        out_v[pl.ds(r0, 16)] = acc

    for c in range(CAT):
        for q in range(RPW // GQ):
            pltpu.make_async_copy(
                scores_hbm.at[idx_v.at[c, pl.ds(q * GQ, GQ)]],
                g_v.at[c, q], sem).wait()
    for m in range(RPW // 16):
        r0 = m * 16
        q, p = divmod(r0, GQ)
        acc = out_v[pl.ds(r0, 16)]
        for c in range(CAT):
            acc = acc + g_v[c, q, pl.ds(p, 16)]
        out_v[pl.ds(r0, 16)] = 1.0 / (1.0 + jnp.exp(-acc))

    pltpu.sync_copy(out_v, out_hbm.at[pl.ds(base, RPW)])


_sc_call = functools.partial(
    pl.kernel,
    out_type=jax.ShapeDtypeStruct((B,), jnp.float32),
    mesh=plsc.VectorSubcoreMesh(core_axis_name="c", subcore_axis_name="s"),
    compiler_params=pltpu.CompilerParams(needs_layout_passes=False),
    scratch_types=[
        pltpu.VMEM((CAT, RPW), jnp.int32),
        pltpu.VMEM((CAT, RPW // GQ, GQ), jnp.float32),
        pltpu.VMEM((CONT, RPW), jnp.float32),
        pltpu.VMEM((CONT + 1, 16), jnp.float32),
        pltpu.VMEM((RPW,), jnp.float32),
        pltpu.SemaphoreType.DMA,
    ],
)(_sc_body)


def kernel(categorical_features, continous_features, emb_table, W, b):
    wcat = W[:CAT * D].reshape(CAT, D)
    scores = _tc_call(wcat, emb_table.T)

    fidx = categorical_features.T.astype(jnp.int32) + jnp.arange(
        CAT, dtype=jnp.int32)[:, None] * VPAD

    wb = jnp.concatenate([W[CAT * D:, 0], b]).astype(jnp.float32)
    wb = jnp.broadcast_to(wb[:, None], (CONT + 1, 16))

    out = _sc_call(scores.reshape(-1), fidx, continous_features.T, wb)
    return out.reshape(B, 1)
